# baseline (speedup 1.0000x reference)
import functools

import jax
import jax.numpy as jnp
from jax import lax
from jax.experimental import pallas as pl
from jax.experimental.pallas import tpu as pltpu
from jax.experimental.pallas import tpu_sc as plsc

_K = 20
_TQ = 256
_TQC = 512
_KPAD = 32
_NW = 32


def _knn_idx_kernel(ptT_ref, pq_ref, idx_ref):
    ptsT = ptT_ref[0]
    q = pq_ref[0]
    n = ptsT.shape[1]
    tq = q.shape[0]
    bi = pl.program_id(0)

    sq_all = jnp.sum(ptsT * ptsT, axis=0, keepdims=True)
    sq_q = jnp.sum(q * q, axis=1, keepdims=True)
    mm = jnp.dot(q, ptsT, preferred_element_type=jnp.float32)
    dist = (sq_q + sq_all) - 2.0 * mm

    iota = lax.broadcasted_iota(jnp.int32, (tq, n), 1)
    col = lax.broadcasted_iota(jnp.int32, (tq, _KPAD), 1)
    out = jnp.zeros((tq, _KPAD), dtype=jnp.int32)
    for k in range(_K):
        m = jnp.min(dist, axis=1, keepdims=True)
        cand = jnp.where(dist == m, iota, n)
        j = jnp.min(cand, axis=1, keepdims=True)
        dist = jnp.where(iota == j, jnp.inf, dist)
        out = jnp.where(col == k, j + bi * n, out)
    idx_ref[0] = out


def _sc_gather_kernel(pts_hbm, idx_hbm, out_hbm, pts_v, idx_v, out_v):
    nidx = idx_v.shape[0]
    wid = lax.axis_index("s") * 2 + lax.axis_index("c")
    base = wid * nidx
    pltpu.sync_copy(pts_hbm, pts_v)
    pltpu.sync_copy(idx_hbm.at[pl.ds(base, nidx)], idx_v)
    lane = lax.broadcasted_iota(jnp.int32, (16,), 0)

    def body(i, carry):
        idx16 = idx_v[pl.ds(i * 16, 16)]
        src = idx16 * 3
        dst = lane * 3 + i * 48
        for c in range(3):
            v = plsc.load_gather(pts_v, [src + c])
            plsc.store_scatter(out_v, [dst + c], v)
        return carry

    lax.fori_loop(0, nidx // 16, body, 0)
    pltpu.sync_copy(out_v, out_hbm.at[pl.ds(base * 3, nidx * 3)])


def _mlp_kernel(feat_ref, pq_ref, W1_ref, b1_ref, W2_ref, b2_ref, out_ref):
    featK = feat_ref[...]
    q = pq_ref[0]
    tq, c = q.shape
    f1 = W1_ref.shape[1]
    f2 = W2_ref.shape[1]

    cterm = (jnp.dot(q, W1_ref[c:, :], preferred_element_type=jnp.float32)
             + b1_ref[0:1, :])
    qrep = jnp.broadcast_to(q[:, None, :], (tq, _K, c)).reshape(tq * _K, c)
    crep = jnp.broadcast_to(cterm[:, None, :], (tq, _K, f1)).reshape(tq * _K, f1)

    rel = featK - qrep
    h1 = jax.nn.relu(
        jnp.dot(rel, W1_ref[:c, :], preferred_element_type=jnp.float32) + crep)
    h2 = jax.nn.relu(
        jnp.dot(h1, W2_ref[...], preferred_element_type=jnp.float32)
        + b2_ref[0:1, :])
    out_ref[0] = jnp.max(h2.reshape(tq, _K, f2), axis=1)


_BC = 8


def _run_chunk(points, ptT, W1, b1r, W2, b2r):
    b, n, c = points.shape
    f1 = W1.shape[1]
    f2 = W2.shape[1]

    idxpad = pl.pallas_call(
        _knn_idx_kernel,
        grid=(b, n // _TQ),
        in_specs=[
            pl.BlockSpec((1, c, n), lambda bi, ti: (bi, 0, 0)),
            pl.BlockSpec((1, _TQ, c), lambda bi, ti: (bi, ti, 0)),
        ],
        out_specs=pl.BlockSpec((1, _TQ, _KPAD), lambda bi, ti: (bi, ti, 0)),
        out_shape=jax.ShapeDtypeStruct((b, n, _KPAD), jnp.int32),
        compiler_params=pltpu.CompilerParams(
            dimension_semantics=("parallel", "parallel")),
    )(ptT, points)

    idxf = idxpad[:, :, :_K].reshape(b * n * _K)
    ptsf = points.reshape(b * n * c)

    nidx = (b * n * _K) // _NW
    mesh = plsc.VectorSubcoreMesh(core_axis_name="c", subcore_axis_name="s")
    sc_gather = functools.partial(
        pl.kernel, mesh=mesh,
        out_type=jax.ShapeDtypeStruct((b * n * _K * c,), jnp.float32),
        scratch_types=[
            pltpu.VMEM((b * n * c,), jnp.float32),
            pltpu.VMEM((nidx,), jnp.int32),
            pltpu.VMEM((nidx * c,), jnp.float32),
        ],
        compiler_params=pltpu.CompilerParams(needs_layout_passes=False),
    )(_sc_gather_kernel)
    featf = sc_gather(ptsf, idxf)
    feat = featf.reshape(b * n * _K, c)

    out = pl.pallas_call(
        _mlp_kernel,
        grid=(b, n // _TQC),
        in_specs=[
            pl.BlockSpec((_TQC * _K, c),
                         lambda bi, ti, nb=n // _TQC: (bi * nb + ti, 0)),
            pl.BlockSpec((1, _TQC, c), lambda bi, ti: (bi, ti, 0)),
            pl.BlockSpec((2 * c, f1), lambda bi, ti: (0, 0)),
            pl.BlockSpec((1, f1), lambda bi, ti: (0, 0)),
            pl.BlockSpec((f1, f2), lambda bi, ti: (0, 0)),
            pl.BlockSpec((1, f2), lambda bi, ti: (0, 0)),
        ],
        out_specs=pl.BlockSpec((1, _TQC, f2), lambda bi, ti: (bi, ti, 0)),
        out_shape=jax.ShapeDtypeStruct((b, n, f2), jnp.float32),
        compiler_params=pltpu.CompilerParams(
            dimension_semantics=("parallel", "parallel")),
    )(feat, points, W1, b1r, W2, b2r)
    return out


def kernel(points, W1, b1, W2, b2):
    b, n, c = points.shape
    f1 = W1.shape[1]
    f2 = W2.shape[1]
    ptT = jnp.transpose(points, (0, 2, 1))
    b1r = b1.reshape(1, f1)
    b2r = b2.reshape(1, f2)

    outs = []
    for s in range(0, b, _BC):
        outs.append(_run_chunk(points[s:s + _BC], ptT[s:s + _BC],
                               W1, b1r, W2, b2r))
    return jnp.concatenate(outs, axis=0)


# speedup vs baseline: 1.2031x; 1.2031x over previous
import functools

import jax
import jax.numpy as jnp
from jax import lax
from jax.experimental import pallas as pl
from jax.experimental.pallas import tpu as pltpu
from jax.experimental.pallas import tpu_sc as plsc

_K = 20
_TQ = 256
_TQC = 1024
_KPAD = 32
_NW = 32


def _knn_idx_kernel(ptT_ref, pq_ref, idx_ref):
    ptsT = ptT_ref[0]
    q = pq_ref[0]
    n = ptsT.shape[1]
    tq = q.shape[0]
    bi = pl.program_id(0)

    sq_all = jnp.sum(ptsT * ptsT, axis=0, keepdims=True)
    sq_q = jnp.sum(q * q, axis=1, keepdims=True)
    mm = jnp.dot(q, ptsT, preferred_element_type=jnp.float32)
    dist = (sq_q + sq_all) - 2.0 * mm

    iota = lax.broadcasted_iota(jnp.int32, (tq, n), 1)
    col = lax.broadcasted_iota(jnp.int32, (tq, _KPAD), 1)
    out = jnp.zeros((tq, _KPAD), dtype=jnp.int32)
    for k in range(_K):
        j = jnp.argmin(dist, axis=1).astype(jnp.int32)[:, None]
        dist = jnp.where(iota == j, jnp.inf, dist)
        out = jnp.where(col == k, j + bi * n, out)
    idx_ref[0] = out


def _sc_gather_kernel(pts_hbm, idx_hbm, out_hbm, pts_v, idx_v, out_v):
    nidx = idx_v.shape[0]
    wid = lax.axis_index("s") * 2 + lax.axis_index("c")
    base = wid * nidx
    pltpu.sync_copy(pts_hbm, pts_v)
    pltpu.sync_copy(idx_hbm.at[pl.ds(base, nidx)], idx_v)
    lane = lax.broadcasted_iota(jnp.int32, (16,), 0)

    def body(i, carry):
        idx16 = idx_v[pl.ds(i * 16, 16)]
        src = idx16 * 3
        dst = lane * 3 + i * 48
        for c in range(3):
            v = plsc.load_gather(pts_v, [src + c])
            plsc.store_scatter(out_v, [dst + c], v)
        return carry

    lax.fori_loop(0, nidx // 16, body, 0)
    pltpu.sync_copy(out_v, out_hbm.at[pl.ds(base * 3, nidx * 3)])


def _mlp_kernel(feat_ref, pq_ref, W1_ref, b1_ref, W2_ref, b2_ref, out_ref):
    featK = feat_ref[...]
    q = pq_ref[0]
    tq, c = q.shape
    f1 = W1_ref.shape[1]
    f2 = W2_ref.shape[1]

    cterm = (jnp.dot(q, W1_ref[c:, :], preferred_element_type=jnp.float32)
             + b1_ref[0:1, :])
    qrep = jnp.broadcast_to(q[:, None, :], (tq, _K, c)).reshape(tq * _K, c)
    crep = jnp.broadcast_to(cterm[:, None, :], (tq, _K, f1)).reshape(tq * _K, f1)

    rel = featK - qrep
    h1 = jax.nn.relu(
        jnp.dot(rel, W1_ref[:c, :], preferred_element_type=jnp.float32) + crep)
    h2 = jax.nn.relu(
        jnp.dot(h1, W2_ref[...], preferred_element_type=jnp.float32)
        + b2_ref[0:1, :])
    out_ref[0] = jnp.max(h2.reshape(tq, _K, f2), axis=1)


_BC = 8


def _run_chunk(points, ptT, W1, b1r, W2, b2r):
    b, n, c = points.shape
    f1 = W1.shape[1]
    f2 = W2.shape[1]

    idxpad = pl.pallas_call(
        _knn_idx_kernel,
        grid=(b, n // _TQ),
        in_specs=[
            pl.BlockSpec((1, c, n), lambda bi, ti: (bi, 0, 0)),
            pl.BlockSpec((1, _TQ, c), lambda bi, ti: (bi, ti, 0)),
        ],
        out_specs=pl.BlockSpec((1, _TQ, _KPAD), lambda bi, ti: (bi, ti, 0)),
        out_shape=jax.ShapeDtypeStruct((b, n, _KPAD), jnp.int32),
        compiler_params=pltpu.CompilerParams(
            dimension_semantics=("parallel", "parallel")),
    )(ptT, points)

    idxf = idxpad[:, :, :_K].reshape(b * n * _K)
    ptsf = points.reshape(b * n * c)

    nidx = (b * n * _K) // _NW
    mesh = plsc.VectorSubcoreMesh(core_axis_name="c", subcore_axis_name="s")
    sc_gather = functools.partial(
        pl.kernel, mesh=mesh,
        out_type=jax.ShapeDtypeStruct((b * n * _K * c,), jnp.float32),
        scratch_types=[
            pltpu.VMEM((b * n * c,), jnp.float32),
            pltpu.VMEM((nidx,), jnp.int32),
            pltpu.VMEM((nidx * c,), jnp.float32),
        ],
        compiler_params=pltpu.CompilerParams(needs_layout_passes=False),
    )(_sc_gather_kernel)
    featf = sc_gather(ptsf, idxf)
    feat = featf.reshape(b * n * _K, c)

    out = pl.pallas_call(
        _mlp_kernel,
        grid=(b, n // _TQC),
        in_specs=[
            pl.BlockSpec((_TQC * _K, c),
                         lambda bi, ti, nb=n // _TQC: (bi * nb + ti, 0)),
            pl.BlockSpec((1, _TQC, c), lambda bi, ti: (bi, ti, 0)),
            pl.BlockSpec((2 * c, f1), lambda bi, ti: (0, 0)),
            pl.BlockSpec((1, f1), lambda bi, ti: (0, 0)),
            pl.BlockSpec((f1, f2), lambda bi, ti: (0, 0)),
            pl.BlockSpec((1, f2), lambda bi, ti: (0, 0)),
        ],
        out_specs=pl.BlockSpec((1, _TQC, f2), lambda bi, ti: (bi, ti, 0)),
        out_shape=jax.ShapeDtypeStruct((b, n, f2), jnp.float32),
        compiler_params=pltpu.CompilerParams(
            dimension_semantics=("parallel", "parallel")),
    )(feat, points, W1, b1r, W2, b2r)
    return out


def kernel(points, W1, b1, W2, b2):
    b, n, c = points.shape
    f1 = W1.shape[1]
    f2 = W2.shape[1]
    ptT = jnp.transpose(points, (0, 2, 1))
    b1r = b1.reshape(1, f1)
    b2r = b2.reshape(1, f2)

    outs = []
    for s in range(0, b, _BC):
        outs.append(_run_chunk(points[s:s + _BC], ptT[s:s + _BC],
                               W1, b1r, W2, b2r))
    return jnp.concatenate(outs, axis=0)
